# baseline (speedup 1.0000x reference)
import jax
import jax.numpy as jnp
from jax.experimental import pallas as pl
from jax.experimental.pallas import tpu as pltpu

NUM_MOVIES = 100000
L_OVRV, L_CAST, L_GENRE, L_PC, L_PCO = 200, 50, 5, 5, 3
D_TITLE, D_OVRV, D_DIR, D_CAST, D_GENRE, D_PC, D_PCO, D_NUM = (
    20, 20, 8, 10, 15, 10, 10, 16)


def _body(m_ref, title_h, ovrv_h, dir_h, cast_h, genre_h, pc_h, pco_h, num_h,
          wt_h, wo_h, wd_h, wc_h, wg_h, wp_h, wq_h, out_ref,
          si_o, si_c, si_g, si_p, si_q, si_t, si_d,
          ro, rc, rg, rp, rq, rt, rd, num_v, sem1, semw):
  m = m_ref[0, 0]

  h1 = [
      pltpu.make_async_copy(ovrv_h.at[pl.ds(m, 1)], si_o, sem1.at[0]),
      pltpu.make_async_copy(cast_h.at[pl.ds(m, 1)], si_c, sem1.at[1]),
      pltpu.make_async_copy(genre_h.at[pl.ds(m, 1)], si_g, sem1.at[2]),
      pltpu.make_async_copy(pc_h.at[pl.ds(m, 1)], si_p, sem1.at[3]),
      pltpu.make_async_copy(pco_h.at[pl.ds(m, 1)], si_q, sem1.at[4]),
      pltpu.make_async_copy(title_h.at[pl.ds(m, 1)], si_t, sem1.at[5]),
      pltpu.make_async_copy(dir_h.at[pl.ds(m, 1)], si_d, sem1.at[6]),
      pltpu.make_async_copy(num_h.at[pl.ds(m, 1)], num_v, sem1.at[7]),
  ]
  for h in h1:
    h.start()
  for h in h1:
    h.wait()

  hs = []

  def fire(si, L, w_h, rbuf):
    for j in range(L):
      v = si[0, j]
      h = pltpu.make_async_copy(w_h.at[pl.ds(v, 1)], rbuf.at[pl.ds(j, 1)],
                                semw)
      h.start()
      hs.append(h)

  fire(si_o, L_OVRV, wo_h, ro)
  fire(si_c, L_CAST, wc_h, rc)
  fire(si_g, L_GENRE, wg_h, rg)
  fire(si_p, L_PC, wp_h, rp)
  fire(si_q, L_PCO, wq_h, rq)
  fire(si_t, 1, wt_h, rt)
  fire(si_d, 1, wd_h, rd)
  for h in hs:
    h.wait()

  def place(row, rbuf, D, scale):
    s = jnp.sum(rbuf[...], axis=0, keepdims=True)
    if scale != 1.0:
      s = s * jnp.float32(scale)
    out_ref[pl.ds(row, 1), pl.ds(0, D)] = s

  place(0, rt, D_TITLE, 1.0)
  place(1, ro, D_OVRV, 1.0 / L_OVRV)
  place(2, rd, D_DIR, 1.0)
  place(3, rc, D_CAST, 1.0 / L_CAST)
  place(4, rg, D_GENRE, 1.0 / L_GENRE)
  place(5, rp, D_PC, 1.0 / L_PC)
  place(6, rq, D_PCO, 1.0 / L_PCO)
  place(7, num_v, D_NUM, 1.0)


@jax.jit
def _tc_call(m2, title, ovrv, director, cast, genre, pc, pco, num, wt, wo,
             wd, wc, wg, wp, wq):
  out8 = pl.pallas_call(
      _body,
      out_shape=jax.ShapeDtypeStruct((8, 128), jnp.float32),
      in_specs=[pl.BlockSpec(memory_space=pltpu.SMEM)] +
               [pl.BlockSpec(memory_space=pltpu.MemorySpace.HBM)] * 15,
      out_specs=pl.BlockSpec(memory_space=pltpu.VMEM),
      scratch_shapes=[
          pltpu.SMEM((1, L_OVRV), jnp.int32),
          pltpu.SMEM((1, L_CAST), jnp.int32),
          pltpu.SMEM((1, L_GENRE), jnp.int32),
          pltpu.SMEM((1, L_PC), jnp.int32),
          pltpu.SMEM((1, L_PCO), jnp.int32),
          pltpu.SMEM((1, 1), jnp.int32),
          pltpu.SMEM((1, 1), jnp.int32),
          pltpu.VMEM((L_OVRV, D_OVRV), jnp.float32),
          pltpu.VMEM((L_CAST, D_CAST), jnp.float32),
          pltpu.VMEM((L_GENRE, D_GENRE), jnp.float32),
          pltpu.VMEM((L_PC, D_PC), jnp.float32),
          pltpu.VMEM((L_PCO, D_PCO), jnp.float32),
          pltpu.VMEM((1, D_TITLE), jnp.float32),
          pltpu.VMEM((1, D_DIR), jnp.float32),
          pltpu.VMEM((1, D_NUM), jnp.float32),
          pltpu.SemaphoreType.DMA((8,)),
          pltpu.SemaphoreType.DMA,
      ],
  )(m2, title, ovrv, director, cast, genre, pc, pco, num, wt, wo, wd, wc,
    wg, wp, wq)
  return jnp.concatenate(
      (out8[0, :D_TITLE], out8[1, :D_OVRV], out8[2, :D_DIR],
       out8[3, :D_CAST], out8[4, :D_GENRE], out8[5, :D_PC], out8[6, :D_PCO],
       out8[7, :D_NUM]))


def kernel(movie_ids, title, overrview, director, cast, genre,
           production_compaines, production_countries, numeric_movie_data,
           W_title, W_ovrv, W_dir, W_cast, W_genre, W_pc, W_pco):
  m2 = jnp.reshape(jnp.asarray(movie_ids, jnp.int32) - 1, (1, 1))
  title2 = jnp.reshape(title, (NUM_MOVIES, 1))
  dir2 = jnp.reshape(director, (NUM_MOVIES, 1))
  return _tc_call(m2, title2, overrview, dir2, cast, genre,
                  production_compaines, production_countries,
                  numeric_movie_data, W_title, W_ovrv, W_dir, W_cast,
                  W_genre, W_pc, W_pco)
